# Initial kernel scaffold; baseline (speedup 1.0000x reference)
import functools

import jax
import jax.numpy as jnp
from jax import lax
from jax.experimental import pallas as pl
from jax.experimental.pallas import tpu as pltpu
from jax.experimental.pallas import tpu_sc as plsc

_info = plsc.get_sparse_core_info()
_NC, _NS, _L = _info.num_cores, _info.num_subcores, _info.num_lanes
_NW = _NC * _NS

_CDF_PAD = 512


def _sample_kernel(n, chunk, u_hbm, cdf_hbm, out_hbm, cdf_v, u_v, out_v):
    wid = lax.axis_index("s") * _NC + lax.axis_index("c")
    base = wid * chunk
    pltpu.sync_copy(cdf_hbm, cdf_v)
    pltpu.sync_copy(u_hbm.at[pl.ds(base, chunk)], u_v)

    inv_n = 1.0 / n

    def body(i, carry):
        uu = u_v[pl.ds(i * _L, _L)]
        idx = jnp.full((_L,), -1, jnp.int32)
        step = _CDF_PAD // 2
        while step >= 1:
            cand = idx + step
            vals = plsc.load_gather(cdf_v, [cand])
            idx = jnp.where(vals < uu, cand, idx)
            step //= 2
        off = jnp.clip(idx + 1, 0, n - 1)
        c0 = plsc.load_gather(cdf_v, [off])
        c1 = plsc.load_gather(cdf_v, [off + 1])
        du = uu - c0
        den = c1 - c0
        pos = den > 0.0
        du = jnp.where(pos, du / jnp.where(pos, den, jnp.float32(1.0)), du)
        out_v[pl.ds(i * _L, _L)] = (off.astype(jnp.float32) + du) * inv_n
        return carry

    lax.fori_loop(0, chunk // _L, body, 0, unroll=4)
    pltpu.sync_copy(out_v, out_hbm.at[pl.ds(base, chunk)])


def kernel(u, pdf, cdf, func):
    del pdf
    n = func.shape[0]
    b = u.shape[0]
    chunk = b // _NW
    cdf_pad = jnp.concatenate(
        [cdf, jnp.full((_CDF_PAD - cdf.shape[0],), 2.0, jnp.float32)]
    )
    mesh = plsc.VectorSubcoreMesh(core_axis_name="c", subcore_axis_name="s")
    run = pl.kernel(
        functools.partial(_sample_kernel, n, chunk),
        out_type=jax.ShapeDtypeStruct((b,), jnp.float32),
        mesh=mesh,
        scratch_types=[
            pltpu.VMEM((_CDF_PAD,), jnp.float32),
            pltpu.VMEM((chunk,), jnp.float32),
            pltpu.VMEM((chunk,), jnp.float32),
        ],
    )
    return run(u, cdf_pad)


# speedup vs baseline: 455.9560x; 455.9560x over previous
import functools

import jax
import jax.numpy as jnp
from jax import lax
from jax.experimental import pallas as pl
from jax.experimental.pallas import tpu as pltpu
from jax.experimental.pallas import tpu_sc as plsc

_info = plsc.get_sparse_core_info()
_NC, _NS, _L = _info.num_cores, _info.num_subcores, _info.num_lanes
_NW = _NC * _NS

_CDF_PAD = 512


def _sample_kernel(n, chunk, u_hbm, cdf_hbm, out_hbm, cdf_v, u_v, out_v):
    wid = lax.axis_index("s") * _NC + lax.axis_index("c")
    base = wid * chunk
    pltpu.sync_copy(cdf_hbm, cdf_v)
    pltpu.sync_copy(u_hbm.at[pl.ds(base, chunk)], u_v)

    inv_n = 1.0 / n

    def body(i, carry):
        uu = u_v[pl.ds(i * _L, _L)]
        idx = jnp.full((_L,), -1, jnp.int32)
        step = _CDF_PAD // 2
        while step >= 1:
            cand = idx + step
            vals = plsc.load_gather(cdf_v, [cand])
            idx = jnp.where(vals < uu, cand, idx)
            step //= 2
        off = jnp.clip(idx + 1, 0, n - 1)
        c0 = plsc.load_gather(cdf_v, [off])
        c1 = plsc.load_gather(cdf_v, [off + 1])
        du = uu - c0
        den = c1 - c0
        pos = den > 0.0
        du = jnp.where(pos, du / jnp.where(pos, den, jnp.float32(1.0)), du)
        out_v[pl.ds(i * _L, _L)] = (off.astype(jnp.float32) + du) * inv_n
        return carry

    lax.fori_loop(0, chunk // _L, body, 0, unroll=4)
    pltpu.sync_copy(out_v, out_hbm.at[pl.ds(base, chunk)])


def kernel(u, pdf, cdf, func):
    del pdf
    n = func.shape[0]
    b = u.shape[0]
    chunk = b // _NW
    cdf_pad = jnp.concatenate(
        [cdf, jnp.full((_CDF_PAD - cdf.shape[0],), 2.0, jnp.float32)]
    )
    mesh = plsc.VectorSubcoreMesh(core_axis_name="c", subcore_axis_name="s")
    run = pl.kernel(
        functools.partial(_sample_kernel, n, chunk),
        out_type=jax.ShapeDtypeStruct((b,), jnp.float32),
        mesh=mesh,
        scratch_types=[
            pltpu.VMEM((_CDF_PAD,), jnp.float32),
            pltpu.VMEM((chunk,), jnp.float32),
            pltpu.VMEM((chunk,), jnp.float32),
        ],
        compiler_params=pltpu.CompilerParams(needs_layout_passes=False),
    )
    return run(u, cdf_pad)


# speedup vs baseline: 1266.6500x; 2.7780x over previous
import functools

import jax
import jax.numpy as jnp
from jax import lax
from jax.experimental import pallas as pl
from jax.experimental.pallas import tpu as pltpu
from jax.experimental.pallas import tpu_sc as plsc

_info = plsc.get_sparse_core_info()
_NC, _NS, _L = _info.num_cores, _info.num_subcores, _info.num_lanes
_NW = _NC * _NS

_VPI = 8


def _sample_kernel(n, chunk, u_hbm, cdf_hbm, out_hbm, cdf_v, u_v, out_v):
    wid = lax.axis_index("s") * _NC + lax.axis_index("c")
    base = wid * chunk
    pltpu.sync_copy(cdf_hbm, cdf_v)
    pltpu.sync_copy(u_hbm.at[pl.ds(base, chunk)], u_v)

    inv_n = jnp.float32(1.0 / n)
    def splat(i):
        return plsc.load_gather(cdf_v, [jnp.full((_L,), i, jnp.int32)])

    c128, c64, c192 = splat(128), splat(64), splat(192)

    def body(i, carry):
        us = [u_v[pl.ds((i * _VPI + j) * _L, _L)] for j in range(_VPI)]
        p1 = [c128 < u for u in us]
        ms = [jnp.where(p, 128, 0).astype(jnp.int32) for p in p1]
        v2 = [jnp.where(p, c192, c64) for p in p1]
        ms = [
            jnp.where(v < u, m + 64, m) for v, u, m in zip(v2, us, ms)
        ]
        for b in (32, 16, 8, 4, 2, 1):
            cand = [m + b for m in ms]
            vals = [plsc.load_gather(cdf_v, [c]) for c in cand]
            ms = [
                jnp.where(v < u, c, m)
                for v, u, c, m in zip(vals, us, cand, ms)
            ]
        offs = [
            jnp.minimum(m + (u > 0.0).astype(jnp.int32), n - 1)
            for m, u in zip(ms, us)
        ]
        c0s = [plsc.load_gather(cdf_v, [o]) for o in offs]
        c1s = [plsc.load_gather(cdf_v, [o + 1]) for o in offs]
        for j in range(_VPI):
            du = us[j] - c0s[j]
            den = c1s[j] - c0s[j]
            pos = den > 0.0
            du = jnp.where(pos, du / jnp.where(pos, den, jnp.float32(1.0)), du)
            out_v[pl.ds((i * _VPI + j) * _L, _L)] = (
                offs[j].astype(jnp.float32) + du
            ) * inv_n
        return carry

    lax.fori_loop(0, chunk // (_L * _VPI), body, 0)
    pltpu.sync_copy(out_v, out_hbm.at[pl.ds(base, chunk)])


def kernel(u, pdf, cdf, func):
    del pdf
    n = func.shape[0]
    b = u.shape[0]
    chunk = b // _NW
    mesh = plsc.VectorSubcoreMesh(core_axis_name="c", subcore_axis_name="s")
    run = pl.kernel(
        functools.partial(_sample_kernel, n, chunk),
        out_type=jax.ShapeDtypeStruct((b,), jnp.float32),
        mesh=mesh,
        scratch_types=[
            pltpu.VMEM((cdf.shape[0],), jnp.float32),
            pltpu.VMEM((chunk,), jnp.float32),
            pltpu.VMEM((chunk,), jnp.float32),
        ],
        compiler_params=pltpu.CompilerParams(needs_layout_passes=False),
    )
    return run(u, cdf)


# speedup vs baseline: 1365.5405x; 1.0781x over previous
import functools

import jax
import jax.numpy as jnp
from jax import lax
from jax.experimental import pallas as pl
from jax.experimental.pallas import tpu as pltpu
from jax.experimental.pallas import tpu_sc as plsc

_info = plsc.get_sparse_core_info()
_NC, _NS, _L = _info.num_cores, _info.num_subcores, _info.num_lanes
_NW = _NC * _NS

_VPI = 8


def _sample_kernel(n, chunk, u_hbm, cdf_hbm, out_hbm, cdf_v, u_v, out_v):
    wid = lax.axis_index("s") * _NC + lax.axis_index("c")
    base = wid * chunk
    pltpu.sync_copy(cdf_hbm, cdf_v)
    pltpu.sync_copy(u_hbm.at[pl.ds(base, chunk)], u_v)

    inv_n = jnp.float32(1.0 / n)
    def splat(i):
        return plsc.load_gather(cdf_v, [jnp.full((_L,), i, jnp.int32)])

    c128, c64, c192 = splat(128), splat(64), splat(192)

    @plsc.parallel_loop(0, chunk // (_L * _VPI), unroll=2)
    def body(i):
        us = [u_v[pl.ds((i * _VPI + j) * _L, _L)] for j in range(_VPI)]
        p1 = [c128 < u for u in us]
        ms = [jnp.where(p, 128, 0).astype(jnp.int32) for p in p1]
        v2 = [jnp.where(p, c192, c64) for p in p1]
        ms = [
            jnp.where(v < u, m + 64, m) for v, u, m in zip(v2, us, ms)
        ]
        for b in (32, 16, 8, 4, 2, 1):
            cand = [m + b for m in ms]
            vals = [plsc.load_gather(cdf_v, [c]) for c in cand]
            ms = [
                jnp.where(v < u, c, m)
                for v, u, c, m in zip(vals, us, cand, ms)
            ]
        offs = [
            jnp.minimum(m + (u > 0.0).astype(jnp.int32), n - 1)
            for m, u in zip(ms, us)
        ]
        c0s = [plsc.load_gather(cdf_v, [o]) for o in offs]
        c1s = [plsc.load_gather(cdf_v, [o + 1]) for o in offs]
        for j in range(_VPI):
            du = us[j] - c0s[j]
            den = c1s[j] - c0s[j]
            pos = den > 0.0
            du = jnp.where(pos, du / jnp.where(pos, den, jnp.float32(1.0)), du)
            out_v[pl.ds((i * _VPI + j) * _L, _L)] = (
                offs[j].astype(jnp.float32) + du
            ) * inv_n

    pltpu.sync_copy(out_v, out_hbm.at[pl.ds(base, chunk)])


def kernel(u, pdf, cdf, func):
    del pdf
    n = func.shape[0]
    b = u.shape[0]
    chunk = b // _NW
    mesh = plsc.VectorSubcoreMesh(core_axis_name="c", subcore_axis_name="s")
    run = pl.kernel(
        functools.partial(_sample_kernel, n, chunk),
        out_type=jax.ShapeDtypeStruct((b,), jnp.float32),
        mesh=mesh,
        scratch_types=[
            pltpu.VMEM((cdf.shape[0],), jnp.float32),
            pltpu.VMEM((chunk,), jnp.float32),
            pltpu.VMEM((chunk,), jnp.float32),
        ],
        compiler_params=pltpu.CompilerParams(needs_layout_passes=False),
    )
    return run(u, cdf)
